# baseline (speedup 1.0000x reference)
import functools
import math

import jax
import jax.numpy as jnp
from jax import lax
from jax.experimental import pallas as pl
from jax.experimental.pallas import tpu as pltpu
from jax.experimental.pallas import tpu_sc as plsc

D_MODEL = 64
SCALE = math.sqrt(D_MODEL)
NUM_CORES = 2
NUM_SUBCORES = 16
NUM_WORKERS = NUM_CORES * NUM_SUBCORES
BT = 128
LANES = 16


@functools.partial(jax.jit, static_argnames=("tokens", "seq"))
def _emb_lookup(idx_flat, lut, *, tokens, seq):
    tok_w = tokens // NUM_WORKERS
    bt_w = tok_w // BT
    n_chunks = seq * bt_w
    n_pairs = n_chunks // 2
    blk_minor = BT * (D_MODEL // 8)
    mesh = plsc.VectorSubcoreMesh(core_axis_name="c", subcore_axis_name="s")

    @functools.partial(
        pl.kernel,
        out_type=jax.ShapeDtypeStruct((seq, 8, tokens // BT, 8, BT),
                                      jnp.float32),
        mesh=mesh,
        scratch_types=[
            pltpu.VMEM((tok_w * seq,), jnp.int32),
            pltpu.VMEM((tok_w * seq,), jnp.int32),
            pltpu.VMEM((BT, D_MODEL), jnp.float32),
            pltpu.VMEM((BT, D_MODEL), jnp.float32),
            pltpu.VMEM((8, 8, BT), jnp.float32),
            pltpu.VMEM((8, 8, BT), jnp.float32),
            pltpu.SemaphoreType.DMA,
            pltpu.SemaphoreType.DMA,
            pltpu.SemaphoreType.DMA,
            pltpu.SemaphoreType.DMA,
        ],
        compiler_params=pltpu.CompilerParams(
            use_tc_tiling_on_sc=False, needs_layout_passes=False),
    )
    def body(lut_hbm, idx_hbm, out_hbm, idx_all, idx_t, rows0, rows1,
             blk0, blk1, sg0, sg1, so0, so1):
        wid = lax.axis_index("s") * NUM_CORES + lax.axis_index("c")
        lane = lax.iota(jnp.int32, LANES)

        pltpu.sync_copy(idx_hbm.at[pl.ds(wid * (tok_w * seq), tok_w * seq)],
                        idx_all)
        lane_seq = lane * seq

        def s_loop(s, _):
            def jg_loop(jg, _):
                addr = lane_seq + (jg * (LANES * seq) + s)
                idx_t[pl.ds(s * tok_w + jg * LANES, LANES)] = (
                    plsc.load_gather(idx_all, [addr]))
                return 0
            lax.fori_loop(0, tok_w // LANES, jg_loop, 0)
            return 0
        lax.fori_loop(0, seq, s_loop, 0)

        def g_copy(c, rows, sem):
            return pltpu.make_async_copy(
                lut_hbm.at[idx_t.at[pl.ds(c * BT, BT)]], rows, sem)

        def o_copy(c, blk, sem):
            s = c // bt_w
            btg = wid * bt_w + lax.rem(c, bt_w)
            return pltpu.make_async_copy(
                blk, out_hbm.at[s, :, btg, :, :], sem)

        lane_hi = lane >> 3
        lane_lo = lane & 7
        lane_zero = lane * 0

        def transpose_scale(rows, blk):
            @plsc.parallel_loop(0, BT, 1, unroll=8)
            def _(i):
                for dg in range(D_MODEL // LANES):
                    v = rows[i, pl.ds(dg * LANES, LANES)] * SCALE
                    plsc.store_scatter(
                        blk, [lane_hi + dg * 2, lane_lo, lane_zero + i], v)

        g_copy(0, rows0, sg0).start()

        def pair(t, _):
            a = 2 * t

            @pl.when(t > 0)
            def _():
                o_copy(a - 1, blk1, so1).wait()

            g_copy(a + 1, rows1, sg1).start()
            g_copy(a, rows0, sg0).wait()
            transpose_scale(rows0, blk0)
            o_copy(a, blk0, so0).start()

            g_copy(a + 1, rows1, sg1).wait()
            transpose_scale(rows1, blk1)
            o_copy(a + 1, blk1, so1).start()

            @pl.when(t < n_pairs - 1)
            def _():
                o_copy(a, blk0, so0).wait()
                g_copy(a + 2, rows0, sg0).start()

            return 0

        lax.fori_loop(0, n_pairs, pair, 0)
        o_copy(n_chunks - 2, blk0, so0).wait()
        o_copy(n_chunks - 1, blk1, so1).wait()

    return body(lut, idx_flat)


def kernel(x, lut):
    tokens, seq = x.shape
    out = _emb_lookup(x.reshape(tokens * seq), lut, tokens=tokens, seq=seq)
    out = jnp.transpose(out, (2, 4, 0, 1, 3))
    return out.reshape(tokens, seq, D_MODEL)


# speedup vs baseline: 1.2213x; 1.2213x over previous
import functools
import math

import jax
import jax.numpy as jnp
from jax import lax
from jax.experimental import pallas as pl
from jax.experimental.pallas import tpu as pltpu
from jax.experimental.pallas import tpu_sc as plsc

D_MODEL = 64
SCALE = math.sqrt(D_MODEL)
NUM_CORES = 2
NUM_SUBCORES = 16
NUM_WORKERS = NUM_CORES * NUM_SUBCORES
BT = 128
LANES = 16


@functools.partial(jax.jit, static_argnames=("tokens", "seq"))
def _emb_lookup(idx_flat, lut, *, tokens, seq):
    tok_w = tokens // NUM_WORKERS
    bt_w = tok_w // BT
    n_chunks = seq * bt_w
    n_pairs = n_chunks // 2
    blk_minor = BT * (D_MODEL // 8)
    mesh = plsc.VectorSubcoreMesh(core_axis_name="c", subcore_axis_name="s")

    @functools.partial(
        pl.kernel,
        out_type=jax.ShapeDtypeStruct((seq, 8, tokens // BT, 8, BT),
                                      jnp.float32),
        mesh=mesh,
        scratch_types=[
            pltpu.VMEM((tok_w * seq,), jnp.int32),
            pltpu.VMEM((tok_w * seq,), jnp.int32),
            pltpu.VMEM((BT, D_MODEL), jnp.float32),
            pltpu.VMEM((BT, D_MODEL), jnp.float32),
            pltpu.VMEM((8, 8, BT), jnp.float32),
            pltpu.VMEM((8, 8, BT), jnp.float32),
            pltpu.SemaphoreType.DMA,
            pltpu.SemaphoreType.DMA,
            pltpu.SemaphoreType.DMA,
            pltpu.SemaphoreType.DMA,
        ],
        compiler_params=pltpu.CompilerParams(
            use_tc_tiling_on_sc=False, needs_layout_passes=False),
    )
    def body(lut_hbm, idx_hbm, out_hbm, idx_all, idx_t, rows0, rows1,
             blk0, blk1, sg0, sg1, so0, so1):
        wid = lax.axis_index("s") * NUM_CORES + lax.axis_index("c")
        lane = lax.iota(jnp.int32, LANES)

        pltpu.sync_copy(idx_hbm.at[pl.ds(wid * (tok_w * seq), tok_w * seq)],
                        idx_all)
        lane_seq = lane * seq

        def s_loop(s, _):
            def jg_loop(jg, _):
                addr = lane_seq + (jg * (LANES * seq) + s)
                idx_t[pl.ds(s * tok_w + jg * LANES, LANES)] = (
                    plsc.load_gather(idx_all, [addr]))
                return 0
            lax.fori_loop(0, tok_w // LANES, jg_loop, 0)
            return 0
        lax.fori_loop(0, seq, s_loop, 0)

        def g_copy(c, rows, sem):
            return pltpu.make_async_copy(
                lut_hbm.at[idx_t.at[pl.ds(c * BT, BT)]], rows, sem)

        def o_copy(c, blk, sem):
            s = c // bt_w
            btg = wid * bt_w + lax.rem(c, bt_w)
            return pltpu.make_async_copy(
                blk, out_hbm.at[s, :, btg, :, :], sem)

        skew = []
        for k in range(LANES):
            x = lax.rem(lane + k, LANES)
            skew.append((x >> 3, x & 7, x))

        def transpose_scale(rows, blk):
            @plsc.parallel_loop(0, BT, LANES, unroll=2)
            def _(i0):
                i_vec = lane + i0
                for dg in range(D_MODEL // LANES):
                    for k in range(LANES):
                        x_hi, x_lo, x = skew[k]
                        v = plsc.load_gather(rows, [i_vec, x + dg * LANES])
                        plsc.store_scatter(
                            blk, [x_hi + dg * 2, x_lo, i_vec], v * SCALE)

        g_copy(0, rows0, sg0).start()

        def pair(t, _):
            a = 2 * t

            @pl.when(t > 0)
            def _():
                o_copy(a - 1, blk1, so1).wait()

            g_copy(a + 1, rows1, sg1).start()
            g_copy(a, rows0, sg0).wait()
            transpose_scale(rows0, blk0)
            o_copy(a, blk0, so0).start()

            g_copy(a + 1, rows1, sg1).wait()
            transpose_scale(rows1, blk1)
            o_copy(a + 1, blk1, so1).start()

            @pl.when(t < n_pairs - 1)
            def _():
                o_copy(a, blk0, so0).wait()
                g_copy(a + 2, rows0, sg0).start()

            return 0

        lax.fori_loop(0, n_pairs, pair, 0)
        o_copy(n_chunks - 2, blk0, so0).wait()
        o_copy(n_chunks - 1, blk1, so1).wait()

    return body(lut, idx_flat)


def kernel(x, lut):
    tokens, seq = x.shape
    out = _emb_lookup(x.reshape(tokens * seq), lut, tokens=tokens, seq=seq)
    out = jnp.transpose(out, (2, 4, 0, 1, 3))
    return out.reshape(tokens, seq, D_MODEL)


# speedup vs baseline: 1.6761x; 1.3724x over previous
import functools
import math

import jax
import jax.numpy as jnp
from jax import lax
from jax.experimental import pallas as pl
from jax.experimental.pallas import tpu as pltpu
from jax.experimental.pallas import tpu_sc as plsc

D_MODEL = 64
SCALE = math.sqrt(D_MODEL)
NUM_CORES = 2
NUM_SUBCORES = 16
NUM_WORKERS = NUM_CORES * NUM_SUBCORES
BT = 128
LANES = 16


@functools.partial(jax.jit, static_argnames=("tokens", "seq"))
def _emb_lookup(idx_flat, lut, *, tokens, seq):
    tok_w = tokens // NUM_WORKERS
    bt_w = tok_w // BT
    n_chunks = seq * bt_w
    n_pairs = n_chunks // 2
    blk_minor = BT * (D_MODEL // 8)
    mesh = plsc.VectorSubcoreMesh(core_axis_name="c", subcore_axis_name="s")

    @functools.partial(
        pl.kernel,
        out_type=jax.ShapeDtypeStruct((seq, 8, tokens // BT, 8, BT),
                                      jnp.float32),
        mesh=mesh,
        scratch_types=[
            pltpu.VMEM((tok_w * seq,), jnp.int32),
            pltpu.VMEM((tok_w * seq,), jnp.int32),
            pltpu.VMEM((BT, D_MODEL), jnp.float32),
            pltpu.VMEM((BT, D_MODEL), jnp.float32),
            pltpu.VMEM((8, 8, BT + 1), jnp.float32),
            pltpu.VMEM((8, 8, BT + 1), jnp.float32),
            pltpu.SemaphoreType.DMA,
            pltpu.SemaphoreType.DMA,
            pltpu.SemaphoreType.DMA,
            pltpu.SemaphoreType.DMA,
        ],
        compiler_params=pltpu.CompilerParams(
            use_tc_tiling_on_sc=False, needs_layout_passes=False),
    )
    def body(lut_hbm, idx_hbm, out_hbm, idx_all, idx_t, rows0, rows1,
             blk0, blk1, sg0, sg1, so0, so1):
        wid = lax.axis_index("s") * NUM_CORES + lax.axis_index("c")
        lane = lax.iota(jnp.int32, LANES)

        pltpu.sync_copy(idx_hbm.at[pl.ds(wid * (tok_w * seq), tok_w * seq)],
                        idx_all)
        lane_seq = lane * seq

        def s_loop(s, _):
            def jg_loop(jg, _):
                addr = lane_seq + (jg * (LANES * seq) + s)
                idx_t[pl.ds(s * tok_w + jg * LANES, LANES)] = (
                    plsc.load_gather(idx_all, [addr]))
                return 0
            lax.fori_loop(0, tok_w // LANES, jg_loop, 0)
            return 0
        lax.fori_loop(0, seq, s_loop, 0)

        def g_copy(c, rows, sem):
            return pltpu.make_async_copy(
                lut_hbm.at[idx_t.at[pl.ds(c * BT, BT)]], rows, sem)

        def o_copy(c, blk, sem):
            s = c // bt_w
            btg = wid * bt_w + lax.rem(c, bt_w)
            return pltpu.make_async_copy(
                blk.at[:, :, pl.ds(0, BT)], out_hbm.at[s, :, btg, :, :], sem)

        lane_hi = lane >> 3
        lane_lo = lane & 7

        def transpose_scale(rows, blk):
            @plsc.parallel_loop(0, BT, 1, unroll=8)
            def _(i):
                i_vec = lane * 0 + i
                for dg in range(D_MODEL // LANES):
                    v = rows[i, pl.ds(dg * LANES, LANES)] * SCALE
                    plsc.store_scatter(
                        blk, [lane_hi + dg * 2, lane_lo, i_vec], v)

        g_copy(0, rows0, sg0).start()

        def pair(t, _):
            a = 2 * t

            @pl.when(t > 0)
            def _():
                o_copy(a - 1, blk1, so1).wait()

            g_copy(a + 1, rows1, sg1).start()
            g_copy(a, rows0, sg0).wait()
            transpose_scale(rows0, blk0)
            o_copy(a, blk0, so0).start()

            g_copy(a + 1, rows1, sg1).wait()
            transpose_scale(rows1, blk1)
            o_copy(a + 1, blk1, so1).start()

            @pl.when(t < n_pairs - 1)
            def _():
                o_copy(a, blk0, so0).wait()
                g_copy(a + 2, rows0, sg0).start()

            return 0

        lax.fori_loop(0, n_pairs, pair, 0)
        o_copy(n_chunks - 2, blk0, so0).wait()
        o_copy(n_chunks - 1, blk1, so1).wait()

    return body(lut, idx_flat)


def kernel(x, lut):
    tokens, seq = x.shape
    out = _emb_lookup(x.reshape(tokens * seq), lut, tokens=tokens, seq=seq)
    out = jnp.transpose(out, (2, 4, 0, 1, 3))
    return out.reshape(tokens, seq, D_MODEL)
